# Initial kernel scaffold; baseline (speedup 1.0000x reference)
import jax
import jax.numpy as jnp
from jax.experimental import pallas as pl


def kernel(x_title, x_label, edge_label_index):
    raise NotImplementedError("write your pallas kernel here")


# speedup vs baseline: 2.5705x; 2.5705x over previous
import functools

import jax
import jax.numpy as jnp
from jax import lax
from jax.experimental import pallas as pl
from jax.experimental.pallas import tpu as pltpu
from jax.experimental.pallas import tpu_sc as plsc

_NC = 2
_NS = 16
_NW = _NC * _NS
_L = 16


def _bitrev4(i: int) -> int:
    return ((i & 1) << 3) | ((i & 2) << 1) | ((i & 4) >> 1) | ((i & 8) >> 3)


_GATHER_DNUMS = lax.GatherDimensionNumbers(
    offset_dims=(), collapsed_slice_dims=(0,), start_index_map=(0,))


def _take(v, idx):
    return lax.gather(v, idx[:, None], _GATHER_DNUMS, slice_sizes=(1,),
                      mode=lax.GatherScatterMode.PROMISE_IN_BOUNDS)


def _make_sc_kernel(E: int, D: int, C: int):
    e_per_w = E // _NW
    n_chunks = e_per_w // C
    assert e_per_w * _NW == E and n_chunks * C == e_per_w
    assert C % 8 == 0 and C <= 128 and D % _L == 0

    mesh = plsc.VectorSubcoreMesh(core_axis_name="c", subcore_axis_name="s")

    @functools.partial(
        pl.kernel,
        out_type=jax.ShapeDtypeStruct((E,), jnp.float32),
        mesh=mesh,
        scratch_types=[
            pltpu.VMEM((C,), jnp.int32),
            pltpu.VMEM((C,), jnp.int32),
            pltpu.VMEM((C, D), jnp.float32),
            pltpu.VMEM((C, D), jnp.float32),
            pltpu.VMEM((C,), jnp.float32),
            pltpu.SemaphoreType.DMA,
        ],
    )
    def sc_kernel(title, label, heads, tails, out, hidx, tidx, hrows, trows,
                  obuf, sem):
        wid = lax.axis_index("s") * _NC + lax.axis_index("c")
        base = wid * e_per_w
        iota = lax.iota(jnp.int32, _L)
        perms = [iota ^ hw for hw in (8, 4, 2, 1)]
        masks = [(iota & hw) == 0 for hw in (8, 4, 2, 1)]

        def chunk_body(i, carry):
            off = base + i * C
            pltpu.sync_copy(heads.at[pl.ds(off, C)], hidx)
            pltpu.sync_copy(tails.at[pl.ds(off, C)], tidx)
            cp_h = pltpu.async_copy(title.at[hidx], hrows, sem)
            cp_t = pltpu.async_copy(label.at[tidx], trows, sem)
            cp_h.wait()
            cp_t.wait()

            def group_body(j, gcarry):
                eb = j * _L
                accs = []
                for k in range(_L):
                    e = eb + k
                    acc = hrows[e, pl.ds(0, _L)] * trows[e, pl.ds(0, _L)]
                    for g in range(1, D // _L):
                        acc = acc + (hrows[e, pl.ds(g * _L, _L)]
                                     * trows[e, pl.ds(g * _L, _L)])
                    accs.append(acc)
                vs = [accs[_bitrev4(t)] for t in range(_L)]
                for perm, mask in zip(perms, masks):
                    vs = [jnp.where(mask, x + _take(x, perm), y + _take(y, perm))
                          for x, y in zip(vs[0::2], vs[1::2])]
                obuf[pl.ds(eb, _L)] = vs[0]
                return gcarry

            lax.fori_loop(0, C // _L, group_body, 0, unroll=False)
            pltpu.sync_copy(obuf, out.at[pl.ds(off, C)])
            return carry

        lax.fori_loop(0, n_chunks, chunk_body, 0, unroll=False)

    return sc_kernel


def kernel(x_title, x_label, edge_label_index):
    E = edge_label_index.shape[1]
    D = x_title.shape[1]
    idx = edge_label_index.astype(jnp.int32)
    sc = _make_sc_kernel(E, D, C=80)
    return sc(x_title, x_label, idx[0], idx[1])


# speedup vs baseline: 4.3829x; 1.7051x over previous
import functools

import jax
import jax.numpy as jnp
from jax import lax
from jax.experimental import pallas as pl
from jax.experimental.pallas import tpu as pltpu
from jax.experimental.pallas import tpu_sc as plsc

_NC = 2
_NS = 16
_NW = _NC * _NS
_L = 16


def _bitrev4(i: int) -> int:
    return ((i & 1) << 3) | ((i & 2) << 1) | ((i & 4) >> 1) | ((i & 8) >> 3)


_GATHER_DNUMS = lax.GatherDimensionNumbers(
    offset_dims=(), collapsed_slice_dims=(0,), start_index_map=(0,))


def _take(v, idx):
    return lax.gather(v, idx[:, None], _GATHER_DNUMS, slice_sizes=(1,),
                      mode=lax.GatherScatterMode.PROMISE_IN_BOUNDS)


def _make_sc_kernel(E: int, D: int, C: int):
    e_per_w = E // _NW
    n_chunks = e_per_w // C
    assert e_per_w * _NW == E and n_chunks * C == e_per_w
    assert C % _L == 0 and C <= 128 and D % _L == 0

    mesh = plsc.VectorSubcoreMesh(core_axis_name="c", subcore_axis_name="s")

    @functools.partial(
        pl.kernel,
        out_type=jax.ShapeDtypeStruct((E,), jnp.float32),
        mesh=mesh,
        scratch_types=[
            pltpu.VMEM((n_chunks, C), jnp.int32),
            pltpu.VMEM((n_chunks, C), jnp.int32),
            pltpu.VMEM((C, D), jnp.float32),
            pltpu.VMEM((C, D), jnp.float32),
            pltpu.VMEM((C, D), jnp.float32),
            pltpu.VMEM((C, D), jnp.float32),
            pltpu.VMEM((e_per_w,), jnp.float32),
            pltpu.SemaphoreType.DMA,
            pltpu.SemaphoreType.DMA,
        ],
    )
    def sc_kernel(title, label, heads, tails, out, hidx, tidx,
                  hrows0, hrows1, trows0, trows1, obuf, sem0, sem1):
        wid = lax.axis_index("s") * _NC + lax.axis_index("c")
        hrows = (hrows0, hrows1)
        trows = (trows0, trows1)
        sems = (sem0, sem1)
        iota = lax.iota(jnp.int32, _L)
        perms = [iota ^ hw for hw in (8, 4, 2, 1)]
        masks = [(iota & hw) == 0 for hw in (8, 4, 2, 1)]

        pltpu.sync_copy(heads.at[wid], hidx)
        pltpu.sync_copy(tails.at[wid], tidx)

        def fire(cur, slot):
            pltpu.async_copy(title.at[hidx.at[cur]], hrows[slot], sems[slot])
            pltpu.async_copy(label.at[tidx.at[cur]], trows[slot], sems[slot])

        def drain(slot):
            pltpu.make_async_copy(title.at[hidx.at[0]], hrows[slot],
                                  sems[slot]).wait()
            pltpu.make_async_copy(label.at[tidx.at[0]], trows[slot],
                                  sems[slot]).wait()

        def compute(cur, slot):
            obase = cur * C

            def group_body(j, gcarry):
                eb = j * _L
                accs = []
                for k in range(_L):
                    e = eb + k
                    acc = hrows[slot][e, pl.ds(0, _L)] * trows[slot][e, pl.ds(0, _L)]
                    for g in range(1, D // _L):
                        acc = acc + (hrows[slot][e, pl.ds(g * _L, _L)]
                                     * trows[slot][e, pl.ds(g * _L, _L)])
                    accs.append(acc)
                vs = [accs[_bitrev4(t)] for t in range(_L)]
                for perm, mask in zip(perms, masks):
                    vs = [jnp.where(mask, x + _take(x, perm), y + _take(y, perm))
                          for x, y in zip(vs[0::2], vs[1::2])]
                obuf[pl.ds(obase + eb, _L)] = vs[0]
                return gcarry

            lax.fori_loop(0, C // _L, group_body, 0, unroll=False)

        fire(0, 0)

        def pair_body(i, carry):
            for b in range(2):
                cur = 2 * i + b
                drain(b)

                @pl.when(cur + 1 < n_chunks)
                def _():
                    fire(cur + 1, 1 - b)

                compute(cur, b)
            return carry

        lax.fori_loop(0, n_chunks // 2, pair_body, 0, unroll=False)
        if n_chunks % 2:
            drain(0)
            compute(n_chunks - 1, 0)

        pltpu.sync_copy(obuf, out.at[pl.ds(wid * e_per_w, e_per_w)])

    return sc_kernel


def kernel(x_title, x_label, edge_label_index):
    E = edge_label_index.shape[1]
    D = x_title.shape[1]
    C = 80
    e_per_w = E // _NW
    idx = edge_label_index.astype(jnp.int32)
    heads = idx[0].reshape(_NW, e_per_w // C, C)
    tails = idx[1].reshape(_NW, e_per_w // C, C)
    sc = _make_sc_kernel(E, D, C)
    return sc(x_title, x_label, heads, tails)


# speedup vs baseline: 7.2781x; 1.6605x over previous
import functools

import jax
import jax.numpy as jnp
from jax import lax
from jax.experimental import pallas as pl
from jax.experimental.pallas import tpu as pltpu
from jax.experimental.pallas import tpu_sc as plsc

_NC = 2
_NS = 16
_NW = _NC * _NS
_L = 16


def _bitrev4(i: int) -> int:
    return ((i & 1) << 3) | ((i & 2) << 1) | ((i & 4) >> 1) | ((i & 8) >> 3)


_GATHER_DNUMS = lax.GatherDimensionNumbers(
    offset_dims=(), collapsed_slice_dims=(0,), start_index_map=(0,))


def _take(v, idx):
    return lax.gather(v, idx[:, None], _GATHER_DNUMS, slice_sizes=(1,),
                      mode=lax.GatherScatterMode.PROMISE_IN_BOUNDS)


def _make_sc_kernel(E: int, D: int, C: int):
    e_per_w = E // _NW
    n_chunks = e_per_w // C
    assert e_per_w * _NW == E and n_chunks * C == e_per_w
    assert C % _L == 0 and C <= 128 and D % _L == 0

    mesh = plsc.VectorSubcoreMesh(core_axis_name="c", subcore_axis_name="s")

    @functools.partial(
        pl.kernel,
        out_type=jax.ShapeDtypeStruct((E,), jnp.float32),
        mesh=mesh,
        scratch_types=[
            pltpu.VMEM((n_chunks, C), jnp.int32),
            pltpu.VMEM((n_chunks, C), jnp.int32),
            pltpu.VMEM((C, D), jnp.float32),
            pltpu.VMEM((C, D), jnp.float32),
            pltpu.VMEM((C, D), jnp.float32),
            pltpu.VMEM((C, D), jnp.float32),
            pltpu.VMEM((e_per_w,), jnp.float32),
            pltpu.VMEM((C, _L), jnp.float32),
            pltpu.SemaphoreType.DMA,
            pltpu.SemaphoreType.DMA,
        ],
    )
    def sc_kernel(title, label, heads, tails, out, hidx, tidx,
                  hrows0, hrows1, trows0, trows1, obuf, pbuf, sem0, sem1):
        wid = lax.axis_index("s") * _NC + lax.axis_index("c")
        hrows = (hrows0, hrows1)
        trows = (trows0, trows1)
        sems = (sem0, sem1)
        iota = lax.iota(jnp.int32, _L)
        perms = [iota ^ hw for hw in (8, 4, 2, 1)]
        masks = [(iota & hw) == 0 for hw in (8, 4, 2, 1)]

        pltpu.sync_copy(heads.at[wid], hidx)
        pltpu.sync_copy(tails.at[wid], tidx)

        def fire(cur, slot):
            pltpu.async_copy(title.at[hidx.at[cur]], hrows[slot], sems[slot])
            pltpu.async_copy(label.at[tidx.at[cur]], trows[slot], sems[slot])

        def drain(slot):
            pltpu.make_async_copy(title.at[hidx.at[0]], hrows[slot],
                                  sems[slot]).wait()
            pltpu.make_async_copy(label.at[tidx.at[0]], trows[slot],
                                  sems[slot]).wait()

        def compute(cur, slot):
            obase = cur * C

            def edge_body(e, ecarry):
                acc_a = (hrows[slot][e, pl.ds(0, _L)]
                         * trows[slot][e, pl.ds(0, _L)])
                acc_b = (hrows[slot][e, pl.ds(_L, _L)]
                         * trows[slot][e, pl.ds(_L, _L)])
                for g in range(2, D // _L, 2):
                    acc_a = acc_a + (hrows[slot][e, pl.ds(g * _L, _L)]
                                     * trows[slot][e, pl.ds(g * _L, _L)])
                    acc_b = acc_b + (hrows[slot][e, pl.ds((g + 1) * _L, _L)]
                                     * trows[slot][e, pl.ds((g + 1) * _L, _L)])
                pbuf[e, pl.ds(0, _L)] = acc_a + acc_b
                return ecarry

            lax.fori_loop(0, C, edge_body, 0, unroll=False)

            def group_body(j, gcarry):
                eb = j * _L
                stack = []
                for t in range(_L):
                    v, lvl = pbuf[eb + _bitrev4(t), pl.ds(0, _L)], 0
                    while stack and stack[-1][0] == lvl:
                        _, x = stack.pop()
                        y, perm, mask = v, perms[lvl], masks[lvl]
                        v = jnp.where(mask, x + _take(x, perm),
                                      y + _take(y, perm))
                        lvl += 1
                    stack.append((lvl, v))
                obuf[pl.ds(obase + eb, _L)] = stack[0][1]
                return gcarry

            lax.fori_loop(0, C // _L, group_body, 0, unroll=False)

        fire(0, 0)

        def pair_body(i, carry):
            for b in range(2):
                cur = 2 * i + b
                drain(b)

                @pl.when(cur + 1 < n_chunks)
                def _():
                    fire(cur + 1, 1 - b)

                compute(cur, b)
            return carry

        lax.fori_loop(0, n_chunks // 2, pair_body, 0, unroll=False)
        if n_chunks % 2:
            drain(0)
            compute(n_chunks - 1, 0)

        pltpu.sync_copy(obuf, out.at[pl.ds(wid * e_per_w, e_per_w)])

    return sc_kernel


def kernel(x_title, x_label, edge_label_index):
    E = edge_label_index.shape[1]
    D = x_title.shape[1]
    C = 80
    e_per_w = E // _NW
    idx = edge_label_index.astype(jnp.int32)
    heads = idx[0].reshape(_NW, e_per_w // C, C)
    tails = idx[1].reshape(_NW, e_per_w // C, C)
    sc = _make_sc_kernel(E, D, C)
    return sc(x_title, x_label, heads, tails)
